# Initial kernel scaffold; baseline (speedup 1.0000x reference)
import jax
import jax.numpy as jnp
from jax.experimental import pallas as pl


def kernel(x, node_labels, Param_W, Param_b):
    raise NotImplementedError("write your pallas kernel here")


# speedup vs baseline: 508.3690x; 508.3690x over previous
import functools

import jax
import jax.numpy as jnp
from jax import lax
from jax.experimental import pallas as pl
from jax.experimental.pallas import tpu as pltpu
from jax.experimental.pallas import tpu_sc as plsc

N = 100000
D = 16
L = 100000
O = 64
C = 1

NC = 2
NS = 16

CHUNK = 128
NCHUNK = 50
PW = CHUNK * NCHUNK
NPAD = PW * NS

LB = 4096
KSTEPS = 25
LPAD = LB * KSTEPS
NPASS = 5
RANGE = LPAD // (NC * NPASS)
DUMP = CHUNK
STRIPE = RANGE // NS
NGROUP = 5
GSZ = NCHUNK // NGROUP


def _sc_segment_sum(x_w, idx_w, zeros_r):
    mesh = plsc.VectorSubcoreMesh(
        core_axis_name="c", subcore_axis_name="s",
        num_cores=NC, num_subcores=NS)

    @functools.partial(
        pl.kernel,
        out_type=jax.ShapeDtypeStruct((LPAD, D), jnp.float32),
        mesh=mesh,
        scratch_types=[
            pltpu.VMEM((NCHUNK, CHUNK), jnp.int32),
            pltpu.VMEM((NCHUNK, CHUNK, D), jnp.float32),
            pltpu.VMEM_SHARED((RANGE + DUMP, D), jnp.float32),
            pltpu.SemaphoreType.DMA,
        ],
        compiler_params=pltpu.CompilerParams(use_tc_tiling_on_sc=False),
    )
    def k(x_hbm, idx_hbm, zeros_hbm, out_hbm, idx_v, x_v, s_sh, sem):
        c = lax.axis_index("c")
        s = lax.axis_index("s")
        pltpu.sync_copy(x_hbm.at[s], x_v)

        for p in range(NPASS):
            blk = c * NPASS + p
            pltpu.sync_copy(zeros_hbm, s_sh.at[pl.ds(s * STRIPE, STRIPE)])
            pltpu.sync_copy(idx_hbm.at[c, p, s], idx_v)
            plsc.subcore_barrier()

            for g in range(NGROUP):
                @pl.loop(g * GSZ, (g + 1) * GSZ)
                def _fire(j):
                    pltpu.async_copy(x_v.at[j], s_sh.at[idx_v.at[j]], sem,
                                     add=True)

                @pl.loop(g * GSZ, (g + 1) * GSZ)
                def _drain(j):
                    pltpu.make_async_copy(x_v.at[j], s_sh.at[idx_v.at[j]],
                                          sem).wait()

            plsc.subcore_barrier()
            pltpu.sync_copy(
                s_sh.at[pl.ds(s * STRIPE, STRIPE)],
                out_hbm.at[pl.ds(blk * RANGE + s * STRIPE, STRIPE)])

    return k(x_w, idx_w, zeros_r)


def _tc_matmul_body(w_ref, s_ref, b_ref, o_ref):
    kstep = pl.program_id(0)

    @pl.when(kstep == 0)
    def _():
        o_ref[...] = b_ref[...]

    w = w_ref[...]
    col = lax.broadcasted_iota(jnp.int32, (1, LB), 1) + kstep * LB
    w = jnp.where(col < L, w, 0.0)
    o_ref[...] += jnp.dot(w, s_ref[...], preferred_element_type=jnp.float32)


def kernel(x, node_labels, Param_W, Param_b):
    x = x.astype(jnp.float32)
    labels = node_labels.astype(jnp.int32)

    x_p = jnp.pad(x, ((0, NPAD - N), (0, 0)))
    lab_p = jnp.pad(labels, (0, NPAD - N), constant_values=-1)
    pos = jnp.arange(NPAD, dtype=jnp.int32) % DUMP
    blk = lab_p // RANGE
    rel = lab_p - blk * RANGE
    tgt = jnp.arange(NC * NPASS, dtype=jnp.int32)[:, None]
    idx_w = jnp.where(blk[None, :] == tgt, rel[None, :],
                      (RANGE + pos)[None, :])
    idx_w = idx_w.reshape(NC, NPASS, NS, NCHUNK, CHUNK)

    x_w = x_p.reshape(NS, NCHUNK, CHUNK, D)
    zeros_r = jnp.zeros((STRIPE, D), jnp.float32)

    s_tab = _sc_segment_sum(x_w, idx_w, zeros_r)

    w2 = Param_W.reshape(O, L)
    bias = Param_b.reshape(O, D).astype(jnp.float32)

    out = pl.pallas_call(
        _tc_matmul_body,
        grid=(KSTEPS,),
        in_specs=[
            pl.BlockSpec((O, LB), lambda k: (0, k)),
            pl.BlockSpec((LB, D), lambda k: (k, 0)),
            pl.BlockSpec((O, D), lambda k: (0, 0)),
        ],
        out_specs=pl.BlockSpec((O, D), lambda k: (0, 0)),
        out_shape=jax.ShapeDtypeStruct((O, D), jnp.float32),
        compiler_params=pltpu.CompilerParams(
            dimension_semantics=("arbitrary",)),
    )(w2, s_tab, bias)

    return out.reshape(C, O, D)
